# baseline (speedup 1.0000x reference)
import jax
import jax.numpy as jnp
from jax.experimental import pallas as pl
from jax.experimental.pallas import tpu as pltpu

F32 = jnp.float32
BF16 = jnp.bfloat16

_NT = (((1,), (1,)), ((), ()))
_NN = (((1,), (0,)), ((), ()))
_TT = (((0,), (1,)), ((), ()))


def _silu(v):
    return v * jax.lax.logistic(v)


def _ln(v, g, b, eps=1e-5):
    mu = jnp.mean(v, axis=-1, keepdims=True)
    var = jnp.mean((v - mu) ** 2, axis=-1, keepdims=True)
    return (v - mu) / jnp.sqrt(var + eps) * g + b


def _dg(a, b, dn):
    return jax.lax.dot_general(a, b, dn, preferred_element_type=F32)


def _stage1_kernel(x_ref, wg_ref, wu_ref, wpost_ref, wpre_ref,
                   wr_ref, ag_ref, ab_ref,
                   hidden_ref, pre_ref, adapt_in_ref, adapt_out_ref,
                   logits_ref):
    xb = x_ref[...]
    xb16 = xb.astype(BF16)
    gate = _dg(xb16, wg_ref[...], _NT)
    up = _dg(xb16, wu_ref[...], _NT)
    hid16 = (_silu(gate) * up).astype(BF16)
    hidden_ref[...] = hid16
    ag = ag_ref[...]
    ab = ab_ref[...]
    ao = _dg(hid16, wpost_ref[...], _NT)
    adapt_out_ref[...] = _ln(ao, ag, ab).astype(BF16)
    pr = _dg(xb16, wpre_ref[...], _NT)
    pre_ref[...] = pr.astype(BF16)
    adapt_in_ref[...] = _ln(pr, ag, ab).astype(BF16)
    logits_ref[...] = _dg(xb, wr_ref[...], _NT)


def _mm_kernel(wep_ref, wop_ref, wap_ref, wdown_ref, m_ref, m2_ref):
    m_ref[...] = _dg(wop_ref[...], wep_ref[...], _NN).astype(BF16)
    m2_ref[...] = _dg(wdown_ref[...], wap_ref[...], _NN).astype(BF16)


def _router_weights(logits):
    lg = logits[:, 0:2]
    mg = jnp.max(lg, axis=1, keepdims=True)
    eg = jnp.exp(lg - mg)
    gp = eg / jnp.sum(eg, axis=1, keepdims=True)
    gp0 = gp[:, 0:1]
    gp1 = gp[:, 1:2]
    is_g1 = (gp1 > gp0).astype(F32)
    chosen_w = jnp.maximum(gp0, gp1)

    ll = logits[:, 2:6]
    ml = jnp.max(ll, axis=1, keepdims=True)
    el = jnp.exp(ll - ml)
    lp = el / jnp.sum(el, axis=1, keepdims=True)

    cols = [lp[:, j:j + 1] for j in range(4)]
    masks = []
    for j in range(4):
        rank = jnp.zeros_like(cols[j])
        for m in range(4):
            if m == j:
                continue
            gt = (cols[m] > cols[j]) if m > j else (cols[m] >= cols[j])
            rank = rank + gt.astype(F32)
        masks.append((rank < 2.0).astype(F32))
    sel = [cols[j] * masks[j] for j in range(4)]
    lsum = sel[0] + sel[1] + sel[2] + sel[3]
    inv = chosen_w / (lsum + 1e-7)
    fl = [s * inv for s in sel]

    g0 = 1.0 - is_g1
    ew = [fl[j] * g0 for j in range(4)] + [fl[j] * is_g1 for j in range(4)]
    tw = ew[0] + ew[1] + ew[2] + ew[3] + ew[4] + ew[5] + ew[6] + ew[7]
    return ew, tw


def _stage3_kernel(q_ref, k_ref, v_ref, hidden_ref, wdown_ref, pre_ref,
                   logits_ref, wa_ref, lng_ref, lnb_ref, m_ref, m2_ref,
                   o_ref):
    aw = _dg(q_ref[0], k_ref[0], _NT)
    aw = _silu(jnp.clip(aw, -5.0, 5.0))
    adapt = _dg(aw.astype(BF16), v_ref[0], _NN).astype(BF16)

    ew, tw = _router_weights(logits_ref[0])
    pre = pre_ref[0]
    g = None
    for i in range(8):
        h = _dg(pre, wa_ref[i], _NT)
        h = _ln(h, lng_ref[i:i + 1, :], lnb_ref[i:i + 1, :])
        term = ew[i] * h
        g = term if g is None else g + term

    shared = (_dg(hidden_ref[0], wdown_ref[...], _NT)
              + 0.1 * _dg(adapt, m2_ref[...], _NT))
    o_ref[0] = shared * tw + 0.1 * _dg(g.astype(BF16), m_ref[...], _NT)


def kernel(x, W_up, W_gate, W_down, W_pre, W_post, adapt_g, adapt_b,
           W_adapt_proj, Wa, ln_g, ln_b, W_expert_proj, W_output_proj,
           Wg, We):
    B, S, D = x.shape
    H = W_up.shape[0]
    A = W_pre.shape[0]
    E = Wa.shape[0]
    N = B * S

    xf = x.reshape(N, D)
    wg16 = W_gate.astype(BF16)
    wu16 = W_up.astype(BF16)
    wdown16 = W_down.astype(BF16)
    wpost16 = W_post.astype(BF16)
    wpre16 = W_pre.astype(BF16)
    wr = jnp.concatenate(
        [Wg, We, jnp.zeros((8 - Wg.shape[0] - We.shape[0], D), F32)],
        axis=0)
    ag2 = adapt_g.reshape(1, A)
    ab2 = adapt_b.reshape(1, A)

    BM1 = 512
    hidden, pre, adapt_in, adapt_out, logits = pl.pallas_call(
        _stage1_kernel,
        grid=(N // BM1,),
        in_specs=[
            pl.BlockSpec((BM1, D), lambda i: (i, 0)),
            pl.BlockSpec((H, D), lambda i: (0, 0)),
            pl.BlockSpec((H, D), lambda i: (0, 0)),
            pl.BlockSpec((A, H), lambda i: (0, 0)),
            pl.BlockSpec((A, D), lambda i: (0, 0)),
            pl.BlockSpec((8, D), lambda i: (0, 0)),
            pl.BlockSpec((1, A), lambda i: (0, 0)),
            pl.BlockSpec((1, A), lambda i: (0, 0)),
        ],
        out_specs=[
            pl.BlockSpec((BM1, H), lambda i: (i, 0)),
            pl.BlockSpec((BM1, A), lambda i: (i, 0)),
            pl.BlockSpec((BM1, A), lambda i: (i, 0)),
            pl.BlockSpec((BM1, A), lambda i: (i, 0)),
            pl.BlockSpec((BM1, 8), lambda i: (i, 0)),
        ],
        out_shape=[
            jax.ShapeDtypeStruct((N, H), BF16),
            jax.ShapeDtypeStruct((N, A), BF16),
            jax.ShapeDtypeStruct((N, A), BF16),
            jax.ShapeDtypeStruct((N, A), BF16),
            jax.ShapeDtypeStruct((N, 8), F32),
        ],
        compiler_params=pltpu.CompilerParams(
            dimension_semantics=("parallel",)),
    )(xf, wg16, wu16, wpost16, wpre16, wr, ag2, ab2)

    M, M2 = pl.pallas_call(
        _mm_kernel,
        in_specs=[pl.BlockSpec((H, A), lambda: (0, 0)),
                  pl.BlockSpec((D, H), lambda: (0, 0)),
                  pl.BlockSpec((H, A), lambda: (0, 0)),
                  pl.BlockSpec((D, H), lambda: (0, 0))],
        out_specs=[pl.BlockSpec((D, A), lambda: (0, 0)),
                   pl.BlockSpec((D, A), lambda: (0, 0))],
        out_shape=[jax.ShapeDtypeStruct((D, A), BF16),
                   jax.ShapeDtypeStruct((D, A), BF16)],
    )(W_expert_proj, W_output_proj, W_adapt_proj, W_down)

    ai3 = adapt_in.reshape(B, S, A)
    ao3 = adapt_out.reshape(B, S, A)
    hd3 = hidden.reshape(B, S, H)
    pre3 = pre.reshape(B, S, A)
    lg3 = logits.reshape(B, S, 8)
    BM3 = 512
    out = pl.pallas_call(
        _stage3_kernel,
        grid=(B, S // BM3),
        in_specs=[
            pl.BlockSpec((1, BM3, A), lambda b, i: (b, i, 0)),
            pl.BlockSpec((1, S, A), lambda b, i: (b, 0, 0)),
            pl.BlockSpec((1, S, A), lambda b, i: (b, 0, 0)),
            pl.BlockSpec((1, BM3, H), lambda b, i: (b, i, 0)),
            pl.BlockSpec((D, H), lambda b, i: (0, 0)),
            pl.BlockSpec((1, BM3, A), lambda b, i: (b, i, 0)),
            pl.BlockSpec((1, BM3, 8), lambda b, i: (b, i, 0)),
            pl.BlockSpec((E, A, A), lambda b, i: (0, 0, 0)),
            pl.BlockSpec((E, A), lambda b, i: (0, 0)),
            pl.BlockSpec((E, A), lambda b, i: (0, 0)),
            pl.BlockSpec((D, A), lambda b, i: (0, 0)),
            pl.BlockSpec((D, A), lambda b, i: (0, 0)),
        ],
        out_specs=pl.BlockSpec((1, BM3, D), lambda b, i: (b, i, 0)),
        out_shape=jax.ShapeDtypeStruct((B, S, D), F32),
        compiler_params=pltpu.CompilerParams(
            dimension_semantics=("parallel", "parallel")),
    )(ai3, ao3, ai3, hd3, wdown16, pre3, lg3, Wa.astype(BF16), ln_g,
      ln_b, M, M2)

    return out


# speedup vs baseline: 1.0321x; 1.0321x over previous
import jax
import jax.numpy as jnp
from jax.experimental import pallas as pl
from jax.experimental.pallas import tpu as pltpu

F32 = jnp.float32
BF16 = jnp.bfloat16

_NT = (((1,), (1,)), ((), ()))
_NN = (((1,), (0,)), ((), ()))
_TT = (((0,), (1,)), ((), ()))


def _silu(v):
    return v * jax.lax.logistic(v)


def _ln(v, g, b, eps=1e-5):
    mu = jnp.mean(v, axis=-1, keepdims=True)
    var = jnp.mean((v - mu) ** 2, axis=-1, keepdims=True)
    return (v - mu) / jnp.sqrt(var + eps) * g + b


def _dg(a, b, dn):
    return jax.lax.dot_general(a, b, dn, preferred_element_type=F32)


def _stage1_kernel(x_ref, wg_ref, wu_ref, wpost_ref, wpre_ref,
                   wr_ref, ag_ref, ab_ref,
                   hidden_ref, pre_ref, adapt_in_ref, adapt_out_ref,
                   logits_ref):
    xb = x_ref[...]
    xb16 = xb.astype(BF16)
    gate = _dg(xb16, wg_ref[...], _NT)
    up = _dg(xb16, wu_ref[...], _NT)
    hid16 = (_silu(gate) * up).astype(BF16)
    hidden_ref[...] = hid16
    ag = ag_ref[...]
    ab = ab_ref[...]
    ao = _dg(hid16, wpost_ref[...], _NT)
    adapt_out_ref[...] = _ln(ao, ag, ab).astype(BF16)
    pr = _dg(xb16, wpre_ref[...], _NT)
    pre_ref[...] = pr.astype(BF16)
    adapt_in_ref[...] = _ln(pr, ag, ab).astype(BF16)
    logits_ref[...] = _dg(xb, wr_ref[...], _NT)


def _mm_kernel(wep_ref, wop_ref, wap_ref, wdown_ref, m_ref, m2_ref):
    m_ref[...] = _dg(wop_ref[...], wep_ref[...], _NN).astype(BF16)
    m2_ref[...] = _dg(wdown_ref[...], wap_ref[...], _NN).astype(BF16)


def _router_weights(logits):
    lg = logits[:, 0:2]
    mg = jnp.max(lg, axis=1, keepdims=True)
    eg = jnp.exp(lg - mg)
    gp = eg / jnp.sum(eg, axis=1, keepdims=True)
    gp0 = gp[:, 0:1]
    gp1 = gp[:, 1:2]
    is_g1 = (gp1 > gp0).astype(F32)
    chosen_w = jnp.maximum(gp0, gp1)

    ll = logits[:, 2:6]
    ml = jnp.max(ll, axis=1, keepdims=True)
    el = jnp.exp(ll - ml)
    lp = el / jnp.sum(el, axis=1, keepdims=True)

    cols = [lp[:, j:j + 1] for j in range(4)]
    masks = []
    for j in range(4):
        rank = jnp.zeros_like(cols[j])
        for m in range(4):
            if m == j:
                continue
            gt = (cols[m] > cols[j]) if m > j else (cols[m] >= cols[j])
            rank = rank + gt.astype(F32)
        masks.append((rank < 2.0).astype(F32))
    sel = [cols[j] * masks[j] for j in range(4)]
    lsum = sel[0] + sel[1] + sel[2] + sel[3]
    inv = chosen_w / (lsum + 1e-7)
    fl = [s * inv for s in sel]

    g0 = 1.0 - is_g1
    ew = [fl[j] * g0 for j in range(4)] + [fl[j] * is_g1 for j in range(4)]
    tw = ew[0] + ew[1] + ew[2] + ew[3] + ew[4] + ew[5] + ew[6] + ew[7]
    return ew, tw


def _stage3_kernel(q_ref, k_ref, v_ref, hidden_ref, wdown_ref, pre_ref,
                   logits_ref, wa_ref, lng_ref, lnb_ref, m_ref, m2_ref,
                   o_ref):
    aw = _dg(q_ref[0], k_ref[0], _NT)
    aw = _silu(jnp.clip(aw, -5.0, 5.0))
    adapt = _dg(aw.astype(BF16), v_ref[0], _NN).astype(BF16)

    ew, tw = _router_weights(logits_ref[0])
    pre = pre_ref[0]
    g = None
    for i in range(8):
        h = _dg(pre, wa_ref[i], _NT)
        h = _ln(h, lng_ref[i:i + 1, :], lnb_ref[i:i + 1, :])
        term = ew[i] * h
        g = term if g is None else g + term

    shared = (_dg(hidden_ref[0], wdown_ref[...], _NT)
              + 0.1 * _dg(adapt, m2_ref[...], _NT))
    o_ref[0] = shared * tw + 0.1 * _dg(g.astype(BF16), m_ref[...], _NT)


def kernel(x, W_up, W_gate, W_down, W_pre, W_post, adapt_g, adapt_b,
           W_adapt_proj, Wa, ln_g, ln_b, W_expert_proj, W_output_proj,
           Wg, We):
    B, S, D = x.shape
    H = W_up.shape[0]
    A = W_pre.shape[0]
    E = Wa.shape[0]
    N = B * S

    xf = x.reshape(N, D)
    wg16 = W_gate.astype(BF16)
    wu16 = W_up.astype(BF16)
    wdown16 = W_down.astype(BF16)
    wpost16 = W_post.astype(BF16)
    wpre16 = W_pre.astype(BF16)
    wr = jnp.concatenate(
        [Wg, We, jnp.zeros((8 - Wg.shape[0] - We.shape[0], D), F32)],
        axis=0)
    ag2 = adapt_g.reshape(1, A)
    ab2 = adapt_b.reshape(1, A)

    BM1 = 1024
    hidden, pre, adapt_in, adapt_out, logits = pl.pallas_call(
        _stage1_kernel,
        grid=(N // BM1,),
        in_specs=[
            pl.BlockSpec((BM1, D), lambda i: (i, 0)),
            pl.BlockSpec((H, D), lambda i: (0, 0)),
            pl.BlockSpec((H, D), lambda i: (0, 0)),
            pl.BlockSpec((A, H), lambda i: (0, 0)),
            pl.BlockSpec((A, D), lambda i: (0, 0)),
            pl.BlockSpec((8, D), lambda i: (0, 0)),
            pl.BlockSpec((1, A), lambda i: (0, 0)),
            pl.BlockSpec((1, A), lambda i: (0, 0)),
        ],
        out_specs=[
            pl.BlockSpec((BM1, H), lambda i: (i, 0)),
            pl.BlockSpec((BM1, A), lambda i: (i, 0)),
            pl.BlockSpec((BM1, A), lambda i: (i, 0)),
            pl.BlockSpec((BM1, A), lambda i: (i, 0)),
            pl.BlockSpec((BM1, 8), lambda i: (i, 0)),
        ],
        out_shape=[
            jax.ShapeDtypeStruct((N, H), BF16),
            jax.ShapeDtypeStruct((N, A), BF16),
            jax.ShapeDtypeStruct((N, A), BF16),
            jax.ShapeDtypeStruct((N, A), BF16),
            jax.ShapeDtypeStruct((N, 8), F32),
        ],
        compiler_params=pltpu.CompilerParams(
            dimension_semantics=("parallel",)),
    )(xf, wg16, wu16, wpost16, wpre16, wr, ag2, ab2)

    M, M2 = pl.pallas_call(
        _mm_kernel,
        in_specs=[pl.BlockSpec((H, A), lambda: (0, 0)),
                  pl.BlockSpec((D, H), lambda: (0, 0)),
                  pl.BlockSpec((H, A), lambda: (0, 0)),
                  pl.BlockSpec((D, H), lambda: (0, 0))],
        out_specs=[pl.BlockSpec((D, A), lambda: (0, 0)),
                   pl.BlockSpec((D, A), lambda: (0, 0))],
        out_shape=[jax.ShapeDtypeStruct((D, A), BF16),
                   jax.ShapeDtypeStruct((D, A), BF16)],
    )(W_expert_proj, W_output_proj, W_adapt_proj, W_down)

    ai3 = adapt_in.reshape(B, S, A)
    ao3 = adapt_out.reshape(B, S, A)
    hd3 = hidden.reshape(B, S, H)
    pre3 = pre.reshape(B, S, A)
    lg3 = logits.reshape(B, S, 8)
    BM3 = 1024
    out = pl.pallas_call(
        _stage3_kernel,
        grid=(B, S // BM3),
        in_specs=[
            pl.BlockSpec((1, BM3, A), lambda b, i: (b, i, 0)),
            pl.BlockSpec((1, S, A), lambda b, i: (b, 0, 0)),
            pl.BlockSpec((1, S, A), lambda b, i: (b, 0, 0)),
            pl.BlockSpec((1, BM3, H), lambda b, i: (b, i, 0)),
            pl.BlockSpec((D, H), lambda b, i: (0, 0)),
            pl.BlockSpec((1, BM3, A), lambda b, i: (b, i, 0)),
            pl.BlockSpec((1, BM3, 8), lambda b, i: (b, i, 0)),
            pl.BlockSpec((E, A, A), lambda b, i: (0, 0, 0)),
            pl.BlockSpec((E, A), lambda b, i: (0, 0)),
            pl.BlockSpec((E, A), lambda b, i: (0, 0)),
            pl.BlockSpec((D, A), lambda b, i: (0, 0)),
            pl.BlockSpec((D, A), lambda b, i: (0, 0)),
        ],
        out_specs=pl.BlockSpec((1, BM3, D), lambda b, i: (b, i, 0)),
        out_shape=jax.ShapeDtypeStruct((B, S, D), F32),
        compiler_params=pltpu.CompilerParams(
            dimension_semantics=("parallel", "parallel")),
    )(ai3, ao3, ai3, hd3, wdown16, pre3, lg3, Wa.astype(BF16), ln_g,
      ln_b, M, M2)

    return out
